# Initial kernel scaffold; baseline (speedup 1.0000x reference)
import jax
import jax.numpy as jnp
from jax.experimental import pallas as pl


def kernel(x, adj, W0, b0, W1, b1, W2, b2):
    raise NotImplementedError("write your pallas kernel here")


# speedup vs baseline: 1.1845x; 1.1845x over previous
import jax
import jax.numpy as jnp
from jax.experimental import pallas as pl
from jax.experimental.pallas import tpu as pltpu


def _mm_kernel(x_ref, w_ref, o_ref):
    o_ref[...] = jnp.dot(x_ref[...], w_ref[...],
                         preferred_element_type=jnp.float32)


def _prop_kernel(adj_ref, s_ref, b_ref, o_ref):
    acc = jnp.dot(adj_ref[...], s_ref[...],
                  preferred_element_type=jnp.float32)
    o_ref[...] = jnp.maximum(acc + b_ref[...], 0.0)


def _dec_kernel(za_ref, zb_ref, o_ref):
    p = jax.lax.dot_general(za_ref[...], zb_ref[...],
                            (((1,), (1,)), ((), ())),
                            preferred_element_type=jnp.float32)
    o_ref[...] = jax.nn.sigmoid(p)


def _support(inp, w):
    n, c = inp.shape[0], w.shape[1]
    return pl.pallas_call(
        _mm_kernel,
        out_shape=jax.ShapeDtypeStruct((n, c), jnp.float32),
    )(inp, w)


_BM_PROP = 200


def _propagate(adj, s, b):
    n = adj.shape[0]
    c = s.shape[1]
    return pl.pallas_call(
        _prop_kernel,
        grid=(n // _BM_PROP,),
        in_specs=[
            pl.BlockSpec((_BM_PROP, n), lambda i: (i, 0)),
            pl.BlockSpec((n, c), lambda i: (0, 0)),
            pl.BlockSpec((1, c), lambda i: (0, 0)),
        ],
        out_specs=pl.BlockSpec((_BM_PROP, c), lambda i: (i, 0)),
        out_shape=jax.ShapeDtypeStruct((n, c), jnp.float32),
        compiler_params=pltpu.CompilerParams(
            dimension_semantics=("parallel",)),
    )(adj, s, b)


_BM_DEC = 400


def _decode(z):
    n, k = z.shape
    return pl.pallas_call(
        _dec_kernel,
        grid=(n // _BM_DEC,),
        in_specs=[
            pl.BlockSpec((_BM_DEC, k), lambda i: (i, 0)),
            pl.BlockSpec((n, k), lambda i: (0, 0)),
        ],
        out_specs=pl.BlockSpec((_BM_DEC, n), lambda i: (i, 0)),
        out_shape=jax.ShapeDtypeStruct((n, n), jnp.float32),
        compiler_params=pltpu.CompilerParams(
            dimension_semantics=("parallel",)),
    )(z, z)


def kernel(x, adj, W0, b0, W1, b1, W2, b2):
    W12 = jnp.concatenate([W1, W2], axis=1)
    b12 = jnp.concatenate([b1, b2])[None, :]

    s0 = _support(x, W0)
    h = _propagate(adj, s0, b0[None, :])
    s12 = _support(h, W12)
    ml = _propagate(adj, s12, b12)
    mu = ml[:, :32]
    logvar = ml[:, 32:]
    z = mu
    adj_recon = _decode(z)
    return (adj_recon, z, mu, logvar)
